# baseline (speedup 1.0000x reference)
import functools

import jax
import jax.numpy as jnp
from jax import lax
from jax.experimental import pallas as pl
from jax.experimental.pallas import tpu as pltpu
from jax.experimental.pallas import tpu_sc as plsc

TOP_K = 32
CONSTANT_NO = 90000
PADDING_IDX = 0
DIM = 64

NC, NS, L = 2, 16, 16
NW = NC * NS
CHUNK = 128


def _sc_scores_body(goals_hbm, succ_hbm, ent_hbm, rel_hbm, out_hbm,
                    ga_v, gb_v, pa_v, xa_v, ya_v, pb_v, xb_v, yb_v,
                    ha_v, ra_v, ta_v, hb_v, rb_v, tb_v,
                    succ_v, prod_v, sc_v,
                    sem_ga, sem_gb, sem_a, sem_b):
    n = out_hbm.shape[0]
    t_per_w = n // NW
    n_chunks = t_per_w // CHUNK
    wid = lax.axis_index("s") * NC + lax.axis_index("c")
    wbase = wid * t_per_w
    lane = lax.iota(jnp.int32, L)

    pltpu.sync_copy(succ_hbm.at[pl.ds(wbase, t_per_w)], succ_v)

    def goals_copy(c, g, sem):
        return pltpu.make_async_copy(
            goals_hbm.at[pl.ds(wbase + c * CHUNK, CHUNK)], g, sem)

    def deint(g, p_s, x_s, y_s):
        c0v = jnp.zeros((L,), jnp.int32)
        for j in range(CHUNK // L):
            rows = j * L + lane
            sl = pl.ds(j * L, L)
            p_s[sl] = plsc.load_gather(g, [rows, c0v])
            x_s[sl] = plsc.load_gather(g, [rows, c0v + 1])
            y_s[sl] = plsc.load_gather(g, [rows, c0v + 2])

    def gathers(p_s, x_s, y_s, h, r, t, sem):
        return (pltpu.make_async_copy(ent_hbm.at[x_s], h, sem),
                pltpu.make_async_copy(rel_hbm.at[p_s], r, sem),
                pltpu.make_async_copy(ent_hbm.at[y_s], t, sem))

    def fire(p_s, x_s, y_s, h, r, t, sem):
        for cp in gathers(p_s, x_s, y_s, h, r, t, sem):
            cp.start()

    def drain(p_s, x_s, y_s, h, r, t, sem):
        for cp in gathers(p_s, x_s, y_s, h, r, t, sem):
            cp.wait()

    wrot = wid & (L - 1)

    def compute(c, p_s, x_s, y_s, h, r, t):
        def group_body(g, _):
            for i in range(L):
                row = g * L + i
                for q in range(DIM // L):
                    sl = pl.ds(q * L, L)
                    prod_v[pl.ds((row * DIM) + q * L, L)] = (
                        h[row, sl] * r[row, sl] * t[row, sl])
            rowv = (g * (L * DIM)) + (lane * DIM)
            dbase = lane + wrot
            acc = [jnp.zeros((L,), jnp.float32) for _ in range(4)]
            for d in range(DIM):
                fidx = rowv + ((dbase + d) & (DIM - 1))
                acc[d % 4] = acc[d % 4] + plsc.load_gather(prod_v, [fidx])
            vals = (acc[0] + acc[1]) + (acc[2] + acc[3])
            gsl = pl.ds(g * L, L)
            o = c * CHUNK + g * L
            ground = ((x_s[gsl] <= CONSTANT_NO) & (y_s[gsl] <= CONSTANT_NO)
                      & (p_s[gsl] != PADDING_IDX))
            vals = jnp.where(ground, vals, jnp.zeros((L,), jnp.float32))
            vals = jnp.where(succ_v[pl.ds(o, L)] != 0, vals,
                             jnp.full((L,), -1e9, jnp.float32))
            sc_v[pl.ds(o, L)] = vals
            return ()

        lax.fori_loop(0, CHUNK // L, group_body, ())

    slot_a = (pa_v, xa_v, ya_v, ha_v, ra_v, ta_v, sem_a)
    slot_b = (pb_v, xb_v, yb_v, hb_v, rb_v, tb_v, sem_b)

    goals_copy(0, ga_v, sem_ga).start()
    goals_copy(1, gb_v, sem_gb).start()
    goals_copy(0, ga_v, sem_ga).wait()
    deint(ga_v, pa_v, xa_v, ya_v)
    fire(*slot_a)

    def pair_body(i, _):
        c0 = 2 * i
        goals_copy(c0 + 1, gb_v, sem_gb).wait()
        deint(gb_v, pb_v, xb_v, yb_v)
        fire(*slot_b)

        @pl.when(c0 + 2 < n_chunks)
        def _():
            goals_copy(c0 + 2, ga_v, sem_ga).start()

        drain(*slot_a)
        compute(c0, pa_v, xa_v, ya_v, ha_v, ra_v, ta_v)

        @pl.when(c0 + 2 < n_chunks)
        def _():
            goals_copy(c0 + 2, ga_v, sem_ga).wait()
            deint(ga_v, pa_v, xa_v, ya_v)
            fire(*slot_a)

        @pl.when(c0 + 3 < n_chunks)
        def _():
            goals_copy(c0 + 3, gb_v, sem_gb).start()

        drain(*slot_b)
        compute(c0 + 1, pb_v, xb_v, yb_v, hb_v, rb_v, tb_v)
        return ()

    lax.fori_loop(0, n_chunks // 2, pair_body, ())
    pltpu.sync_copy(sc_v, out_hbm.at[pl.ds(wbase, t_per_w)])


def _sc_scores(goals6, succ, ent_emb, rel_emb):
    n = goals6.shape[0]
    t_per_w = n // NW
    mesh = plsc.VectorSubcoreMesh(core_axis_name="c", subcore_axis_name="s",
                                  num_cores=NC, num_subcores=NS)
    return pl.kernel(
        _sc_scores_body,
        out_type=jax.ShapeDtypeStruct((n,), jnp.float32),
        mesh=mesh,
        compiler_params=pltpu.CompilerParams(needs_layout_passes=False,
                                             use_tc_tiling_on_sc=False),
        scratch_types=[
            pltpu.VMEM((CHUNK, 6), jnp.int32),
            pltpu.VMEM((CHUNK, 6), jnp.int32),
            pltpu.VMEM((CHUNK,), jnp.int32),
            pltpu.VMEM((CHUNK,), jnp.int32),
            pltpu.VMEM((CHUNK,), jnp.int32),
            pltpu.VMEM((CHUNK,), jnp.int32),
            pltpu.VMEM((CHUNK,), jnp.int32),
            pltpu.VMEM((CHUNK,), jnp.int32),
            pltpu.VMEM((CHUNK, DIM), jnp.float32),
            pltpu.VMEM((CHUNK, DIM), jnp.float32),
            pltpu.VMEM((CHUNK, DIM), jnp.float32),
            pltpu.VMEM((CHUNK, DIM), jnp.float32),
            pltpu.VMEM((CHUNK, DIM), jnp.float32),
            pltpu.VMEM((CHUNK, DIM), jnp.float32),
            pltpu.VMEM((t_per_w,), jnp.int32),
            pltpu.VMEM((CHUNK * DIM,), jnp.float32),
            pltpu.VMEM((t_per_w,), jnp.float32),
            pltpu.SemaphoreType.DMA,
            pltpu.SemaphoreType.DMA,
            pltpu.SemaphoreType.DMA,
            pltpu.SemaphoreType.DMA,
        ],
    )(goals6, succ, ent_emb, rel_emb)


def _tc_topk_body(s_ref, succ_ref, out_ref, s_scr, keep_scr):
    rb, kr = s_ref.shape
    col = lax.broadcasted_iota(jnp.int32, (rb, kr), 1)
    s_scr[...] = s_ref[...]
    keep_scr[...] = jnp.zeros((rb, kr), jnp.int32)

    def it(_, carry):
        s = s_scr[...]
        m = jnp.max(s, axis=1, keepdims=True)
        first_idx = jnp.min(jnp.where(s == m, col, kr), axis=1, keepdims=True)
        onehot = col == first_idx
        keep_scr[...] = keep_scr[...] | onehot.astype(jnp.int32)
        s_scr[...] = jnp.where(onehot, jnp.float32(-jnp.inf), s)
        return carry

    lax.fori_loop(0, TOP_K, it, 0)
    out_ref[...] = keep_scr[...] & (succ_ref[...] != 0).astype(jnp.int32)


def _tc_topk(scores2d, succ2d, interpret=False):
    n_rows, kr = scores2d.shape
    rb = 256
    return pl.pallas_call(
        _tc_topk_body,
        grid=(n_rows // rb,),
        in_specs=[pl.BlockSpec((rb, kr), lambda i: (i, 0)),
                  pl.BlockSpec((rb, kr), lambda i: (i, 0))],
        out_specs=pl.BlockSpec((rb, kr), lambda i: (i, 0)),
        out_shape=jax.ShapeDtypeStruct((n_rows, kr), jnp.int32),
        scratch_shapes=[pltpu.VMEM((rb, kr), jnp.float32),
                        pltpu.VMEM((rb, kr), jnp.int32)],
        interpret=interpret,
    )(scores2d, succ2d)


def kernel(rule_goals, rule_success, queries, ent_emb, rel_emb):
    b, s, kr = rule_success.shape
    n = b * s * kr
    goals6 = rule_goals.reshape(n, 6)
    succ = rule_success.reshape(-1).astype(jnp.int32)
    scores = _sc_scores(goals6, succ, ent_emb, rel_emb)
    keep = _tc_topk(scores.reshape(b * s, kr), succ.reshape(b * s, kr))
    return rule_success & (keep != 0).reshape(b, s, kr)


# speedup vs baseline: 1.6453x; 1.6453x over previous
import functools

import jax
import jax.numpy as jnp
from jax import lax
from jax.experimental import pallas as pl
from jax.experimental.pallas import tpu as pltpu
from jax.experimental.pallas import tpu_sc as plsc

TOP_K = 32
CONSTANT_NO = 90000
PADDING_IDX = 0
DIM = 64

NC, NS, L = 2, 16, 16
NW = NC * NS
CHUNK = 128


def _sc_scores_body(comb_hbm, ent_hbm, rel_hbm, out_hbm,
                    idx_v, pc_v, xc_v, yc_v, pos_v,
                    ha_v, ra_v, ta_v, hb_v, rb_v, tb_v, sc_v,
                    sem_a, sem_b):
    t_per_w = comb_hbm.shape[1] // 4
    wid = lax.axis_index("s") * NC + lax.axis_index("c")
    wbase = wid * t_per_w
    lane = lax.iota(jnp.int32, L)
    wrot = wid & (L - 1)

    pltpu.sync_copy(comb_hbm.at[wid], idx_v)

    def scan_group(j, off):
        o = j * L
        p16 = idx_v[pl.ds(o, L)]
        x16 = idx_v[pl.ds(t_per_w + o, L)]
        y16 = idx_v[pl.ds(2 * t_per_w + o, L)]
        s16 = idx_v[pl.ds(3 * t_per_w + o, L)]
        ground = ((x16 <= CONSTANT_NO) & (y16 <= CONSTANT_NO)
                  & (p16 != PADDING_IDX))
        succ = s16 != 0
        keep = ground & succ
        sc_v[pl.ds(o, L)] = jnp.where(succ, jnp.zeros((L,), jnp.float32),
                                      jnp.full((L,), -1e9, jnp.float32))
        ki = keep.astype(jnp.int32)
        pfx = plsc.cumsum(ki) - ki
        cnt = jnp.sum(ki)
        tgt = pfx + off
        plsc.store_scatter(pc_v, [tgt], p16, mask=keep)
        plsc.store_scatter(xc_v, [tgt], x16, mask=keep)
        plsc.store_scatter(yc_v, [tgt], y16, mask=keep)
        plsc.store_scatter(pos_v, [tgt], o + lane, mask=keep)
        return off + cnt

    cnt = lax.fori_loop(0, t_per_w // L, scan_group, jnp.int32(0))

    zero16 = jnp.zeros((L,), jnp.int32)
    for j in range(CHUNK // L):
        tail = cnt + (j * L) + lane
        plsc.store_scatter(pc_v, [tail], zero16)
        plsc.store_scatter(xc_v, [tail], zero16)
        plsc.store_scatter(yc_v, [tail], zero16)
        plsc.store_scatter(pos_v, [tail], t_per_w + (j * L) + lane)

    m_chunks = (cnt + (CHUNK - 1)) // CHUNK

    def gathers(c, h, r, t, sem):
        sl = pl.ds(c * CHUNK, CHUNK)
        return (pltpu.make_async_copy(ent_hbm.at[xc_v.at[sl]], h, sem),
                pltpu.make_async_copy(rel_hbm.at[pc_v.at[sl]], r, sem),
                pltpu.make_async_copy(ent_hbm.at[yc_v.at[sl]], t, sem))

    def fire(c, h, r, t, sem):
        for cp in gathers(c, h, r, t, sem):
            cp.start()

    def drain(c, h, r, t, sem):
        for cp in gathers(c, h, r, t, sem):
            cp.wait()

    def compute(c, h, r, t, sem):
        del sem
        def group_body(g, _):
            for i in range(L):
                row = g * L + i
                for q in range(DIM // L):
                    sl = pl.ds(q * L, L)
                    h[row, sl] = h[row, sl] * r[row, sl] * t[row, sl]
            rows = g * L + lane
            dbase = lane + wrot
            acc = [jnp.zeros((L,), jnp.float32) for _ in range(4)]
            for d in range(DIM):
                dvec = (dbase + d) & (DIM - 1)
                acc[d % 4] = acc[d % 4] + plsc.load_gather(h, [rows, dvec])
            vals = (acc[0] + acc[1]) + (acc[2] + acc[3])
            pos16 = pos_v[pl.ds(c * CHUNK + g * L, L)]
            plsc.store_scatter(sc_v, [pos16], vals)
            return ()

        lax.fori_loop(0, CHUNK // L, group_body, ())

    slot_a = (ha_v, ra_v, ta_v, sem_a)
    slot_b = (hb_v, rb_v, tb_v, sem_b)

    @pl.when(m_chunks > 0)
    def _():
        fire(0, *slot_a)

    def pair_body(i, _):
        c0 = 2 * i

        @pl.when(c0 + 1 < m_chunks)
        def _():
            fire(c0 + 1, *slot_b)

        drain(c0, *slot_a)
        compute(c0, *slot_a)

        @pl.when(c0 + 2 < m_chunks)
        def _():
            fire(c0 + 2, *slot_a)

        @pl.when(c0 + 1 < m_chunks)
        def _():
            drain(c0 + 1, *slot_b)
            compute(c0 + 1, *slot_b)

        return ()

    lax.fori_loop(0, (m_chunks + 1) // 2, pair_body, ())
    pltpu.sync_copy(sc_v.at[pl.ds(0, t_per_w)],
                    out_hbm.at[pl.ds(wbase, t_per_w)])


def _sc_scores(comb, ent_emb, rel_emb):
    n = comb.shape[0] * comb.shape[1] // 4
    t_per_w = n // NW
    mesh = plsc.VectorSubcoreMesh(core_axis_name="c", subcore_axis_name="s",
                                  num_cores=NC, num_subcores=NS)
    return pl.kernel(
        _sc_scores_body,
        out_type=jax.ShapeDtypeStruct((n,), jnp.float32),
        mesh=mesh,
        compiler_params=pltpu.CompilerParams(needs_layout_passes=False,
                                             use_tc_tiling_on_sc=False),
        scratch_types=[
            pltpu.VMEM((4 * t_per_w,), jnp.int32),
            pltpu.VMEM((t_per_w + CHUNK,), jnp.int32),
            pltpu.VMEM((t_per_w + CHUNK,), jnp.int32),
            pltpu.VMEM((t_per_w + CHUNK,), jnp.int32),
            pltpu.VMEM((t_per_w + CHUNK,), jnp.int32),
            pltpu.VMEM((CHUNK, DIM), jnp.float32),
            pltpu.VMEM((CHUNK, DIM), jnp.float32),
            pltpu.VMEM((CHUNK, DIM), jnp.float32),
            pltpu.VMEM((CHUNK, DIM), jnp.float32),
            pltpu.VMEM((CHUNK, DIM), jnp.float32),
            pltpu.VMEM((CHUNK, DIM), jnp.float32),
            pltpu.VMEM((t_per_w + CHUNK,), jnp.float32),
            pltpu.SemaphoreType.DMA,
            pltpu.SemaphoreType.DMA,
        ],
    )(comb, ent_emb, rel_emb)


def _tc_topk_body(s_ref, succ_ref, out_ref, s_scr, keep_scr):
    rb, kr = s_ref.shape
    col = lax.broadcasted_iota(jnp.int32, (rb, kr), 1)
    s_scr[...] = s_ref[...]
    keep_scr[...] = jnp.zeros((rb, kr), jnp.int32)

    def it(_, carry):
        s = s_scr[...]
        m = jnp.max(s, axis=1, keepdims=True)
        first_idx = jnp.min(jnp.where(s == m, col, kr), axis=1, keepdims=True)
        onehot = col == first_idx
        keep_scr[...] = keep_scr[...] | onehot.astype(jnp.int32)
        s_scr[...] = jnp.where(onehot, jnp.float32(-jnp.inf), s)
        return carry

    lax.fori_loop(0, TOP_K, it, 0)
    out_ref[...] = keep_scr[...] & (succ_ref[...] != 0).astype(jnp.int32)


def _tc_topk(scores2d, succ2d, interpret=False):
    n_rows, kr = scores2d.shape
    rb = 256
    return pl.pallas_call(
        _tc_topk_body,
        grid=(n_rows // rb,),
        in_specs=[pl.BlockSpec((rb, kr), lambda i: (i, 0)),
                  pl.BlockSpec((rb, kr), lambda i: (i, 0))],
        out_specs=pl.BlockSpec((rb, kr), lambda i: (i, 0)),
        out_shape=jax.ShapeDtypeStruct((n_rows, kr), jnp.int32),
        scratch_shapes=[pltpu.VMEM((rb, kr), jnp.float32),
                        pltpu.VMEM((rb, kr), jnp.int32)],
        interpret=interpret,
    )(scores2d, succ2d)


def kernel(rule_goals, rule_success, queries, ent_emb, rel_emb):
    b, s, kr = rule_success.shape
    n = b * s * kr
    t_per_w = n // NW
    first = rule_goals[:, :, :, 0, :].reshape(-1, 3)
    succ = rule_success.reshape(-1).astype(jnp.int32)
    comb = jnp.stack([first[:, 0].reshape(NW, t_per_w),
                      first[:, 1].reshape(NW, t_per_w),
                      first[:, 2].reshape(NW, t_per_w),
                      succ.reshape(NW, t_per_w)], axis=1).reshape(NW, 4 * t_per_w)
    scores = _sc_scores(comb, ent_emb, rel_emb)
    keep = _tc_topk(scores.reshape(b * s, kr), succ.reshape(b * s, kr))
    return rule_success & (keep != 0).reshape(b, s, kr)


# speedup vs baseline: 1.6514x; 1.0037x over previous
import functools

import jax
import jax.numpy as jnp
from jax import lax
from jax.experimental import pallas as pl
from jax.experimental.pallas import tpu as pltpu
from jax.experimental.pallas import tpu_sc as plsc

TOP_K = 32
CONSTANT_NO = 90000
PADDING_IDX = 0
DIM = 64

NC, NS, L = 2, 16, 16
NW = NC * NS
CHUNK = 128


def _sc_scores_body(comb_hbm, ent_hbm, rel_hbm, out_hbm,
                    idx_v, pc_v, xc_v, yc_v, pos_v,
                    ha_v, ra_v, ta_v, hb_v, rb_v, tb_v, sc_v,
                    sem_a, sem_b):
    t_per_w = comb_hbm.shape[1] // 4
    wid = lax.axis_index("s") * NC + lax.axis_index("c")
    wbase = wid * t_per_w
    lane = lax.iota(jnp.int32, L)
    wrot = wid & (L - 1)

    pltpu.sync_copy(comb_hbm.at[wid], idx_v)

    def scan_group(j, off):
        o = j * L
        p16 = idx_v[pl.ds(o, L)]
        x16 = idx_v[pl.ds(t_per_w + o, L)]
        y16 = idx_v[pl.ds(2 * t_per_w + o, L)]
        s16 = idx_v[pl.ds(3 * t_per_w + o, L)]
        ground = ((x16 <= CONSTANT_NO) & (y16 <= CONSTANT_NO)
                  & (p16 != PADDING_IDX))
        succ = s16 != 0
        keep = ground & succ
        sc_v[pl.ds(o, L)] = jnp.where(succ, jnp.zeros((L,), jnp.float32),
                                      jnp.full((L,), -1e9, jnp.float32))
        ki = keep.astype(jnp.int32)
        pfx = plsc.cumsum(ki) - ki
        cnt = jnp.sum(ki)
        tgt = pfx + off
        plsc.store_scatter(pc_v, [tgt], p16, mask=keep)
        plsc.store_scatter(xc_v, [tgt], x16, mask=keep)
        plsc.store_scatter(yc_v, [tgt], y16, mask=keep)
        plsc.store_scatter(pos_v, [tgt], o + lane, mask=keep)
        return off + cnt

    cnt = lax.fori_loop(0, t_per_w // L, scan_group, jnp.int32(0))

    zero16 = jnp.zeros((L,), jnp.int32)
    for j in range(CHUNK // L):
        tail = cnt + (j * L) + lane
        plsc.store_scatter(pc_v, [tail], zero16)
        plsc.store_scatter(xc_v, [tail], zero16)
        plsc.store_scatter(yc_v, [tail], zero16)
        plsc.store_scatter(pos_v, [tail], t_per_w + (j * L) + lane)

    m_chunks = (cnt + (CHUNK - 1)) // CHUNK

    def gathers(c, h, r, t, sem):
        sl = pl.ds(c * CHUNK, CHUNK)
        return (pltpu.make_async_copy(ent_hbm.at[xc_v.at[sl]], h, sem),
                pltpu.make_async_copy(rel_hbm.at[pc_v.at[sl]], r, sem),
                pltpu.make_async_copy(ent_hbm.at[yc_v.at[sl]], t, sem))

    def fire(c, h, r, t, sem):
        for cp in gathers(c, h, r, t, sem):
            cp.start()

    def drain(c, h, r, t, sem):
        for cp in gathers(c, h, r, t, sem):
            cp.wait()

    def compute(c, h, r, t, sem):
        del sem

        def group_body(g, _):
            for i in range(L):
                row = g * L + i
                for q in range(DIM // L):
                    sl = pl.ds(q * L, L)
                    pr = h[row, sl] * r[row, sl] * t[row, sl]
                    idx_v[pl.ds(row * DIM + q * L, L)] = plsc.bitcast(
                        pr, jnp.int32)
            rowv = (g * (L * DIM)) + (lane * DIM)
            dbase = lane + wrot
            acc = [jnp.zeros((L,), jnp.float32) for _ in range(4)]
            for d in range(DIM):
                fidx = rowv + ((dbase + d) & (DIM - 1))
                acc[d % 4] = acc[d % 4] + plsc.bitcast(
                    plsc.load_gather(idx_v, [fidx]), jnp.float32)
            vals = (acc[0] + acc[1]) + (acc[2] + acc[3])
            pos16 = pos_v[pl.ds(c * CHUNK + g * L, L)]
            plsc.store_scatter(sc_v, [pos16], vals)
            return ()

        lax.fori_loop(0, CHUNK // L, group_body, ())

    slot_a = (ha_v, ra_v, ta_v, sem_a)
    slot_b = (hb_v, rb_v, tb_v, sem_b)

    @pl.when(m_chunks > 0)
    def _():
        fire(0, *slot_a)

    def pair_body(i, _):
        c0 = 2 * i

        @pl.when(c0 + 1 < m_chunks)
        def _():
            fire(c0 + 1, *slot_b)

        drain(c0, *slot_a)
        compute(c0, *slot_a)

        @pl.when(c0 + 2 < m_chunks)
        def _():
            fire(c0 + 2, *slot_a)

        @pl.when(c0 + 1 < m_chunks)
        def _():
            drain(c0 + 1, *slot_b)
            compute(c0 + 1, *slot_b)

        return ()

    lax.fori_loop(0, (m_chunks + 1) // 2, pair_body, ())
    pltpu.sync_copy(sc_v.at[pl.ds(0, t_per_w)],
                    out_hbm.at[pl.ds(wbase, t_per_w)])


def _sc_scores(comb, ent_emb, rel_emb):
    n = comb.shape[0] * comb.shape[1] // 4
    t_per_w = n // NW
    mesh = plsc.VectorSubcoreMesh(core_axis_name="c", subcore_axis_name="s",
                                  num_cores=NC, num_subcores=NS)
    return pl.kernel(
        _sc_scores_body,
        out_type=jax.ShapeDtypeStruct((n,), jnp.float32),
        mesh=mesh,
        compiler_params=pltpu.CompilerParams(needs_layout_passes=False,
                                             use_tc_tiling_on_sc=False),
        scratch_types=[
            pltpu.VMEM((4 * t_per_w,), jnp.int32),
            pltpu.VMEM((t_per_w + CHUNK,), jnp.int32),
            pltpu.VMEM((t_per_w + CHUNK,), jnp.int32),
            pltpu.VMEM((t_per_w + CHUNK,), jnp.int32),
            pltpu.VMEM((t_per_w + CHUNK,), jnp.int32),
            pltpu.VMEM((CHUNK, DIM), jnp.float32),
            pltpu.VMEM((CHUNK, DIM), jnp.float32),
            pltpu.VMEM((CHUNK, DIM), jnp.float32),
            pltpu.VMEM((CHUNK, DIM), jnp.float32),
            pltpu.VMEM((CHUNK, DIM), jnp.float32),
            pltpu.VMEM((CHUNK, DIM), jnp.float32),
            pltpu.VMEM((t_per_w + CHUNK,), jnp.float32),
            pltpu.SemaphoreType.DMA,
            pltpu.SemaphoreType.DMA,
        ],
    )(comb, ent_emb, rel_emb)


def _tc_topk_body(s_ref, succ_ref, out_ref, s_scr, keep_scr):
    rb, kr = s_ref.shape
    col = lax.broadcasted_iota(jnp.int32, (rb, kr), 1)
    s_scr[...] = s_ref[...]
    keep_scr[...] = jnp.zeros((rb, kr), jnp.int32)

    def it(_, carry):
        s = s_scr[...]
        m = jnp.max(s, axis=1, keepdims=True)
        first_idx = jnp.min(jnp.where(s == m, col, kr), axis=1, keepdims=True)
        onehot = col == first_idx
        keep_scr[...] = keep_scr[...] | onehot.astype(jnp.int32)
        s_scr[...] = jnp.where(onehot, jnp.float32(-jnp.inf), s)
        return carry

    lax.fori_loop(0, TOP_K, it, 0)
    out_ref[...] = keep_scr[...] & (succ_ref[...] != 0).astype(jnp.int32)


def _tc_topk(scores2d, succ2d, interpret=False):
    n_rows, kr = scores2d.shape
    rb = 256
    return pl.pallas_call(
        _tc_topk_body,
        grid=(n_rows // rb,),
        in_specs=[pl.BlockSpec((rb, kr), lambda i: (i, 0)),
                  pl.BlockSpec((rb, kr), lambda i: (i, 0))],
        out_specs=pl.BlockSpec((rb, kr), lambda i: (i, 0)),
        out_shape=jax.ShapeDtypeStruct((n_rows, kr), jnp.int32),
        scratch_shapes=[pltpu.VMEM((rb, kr), jnp.float32),
                        pltpu.VMEM((rb, kr), jnp.int32)],
        interpret=interpret,
    )(scores2d, succ2d)


def kernel(rule_goals, rule_success, queries, ent_emb, rel_emb):
    b, s, kr = rule_success.shape
    n = b * s * kr
    t_per_w = n // NW
    first = rule_goals[:, :, :, 0, :].reshape(-1, 3)
    succ = rule_success.reshape(-1).astype(jnp.int32)
    comb = jnp.stack([first[:, 0].reshape(NW, t_per_w),
                      first[:, 1].reshape(NW, t_per_w),
                      first[:, 2].reshape(NW, t_per_w),
                      succ.reshape(NW, t_per_w)], axis=1).reshape(NW, 4 * t_per_w)
    scores = _sc_scores(comb, ent_emb, rel_emb)
    keep = _tc_topk(scores.reshape(b * s, kr), succ.reshape(b * s, kr))
    return rule_success & (keep != 0).reshape(b, s, kr)
